# Initial kernel scaffold; baseline (speedup 1.0000x reference)
import jax
import jax.numpy as jnp
from jax.experimental import pallas as pl


def kernel(char_codes, char_to_id):
    raise NotImplementedError("write your pallas kernel here")


# speedup vs baseline: 158.1535x; 158.1535x over previous
import functools

import jax
import jax.numpy as jnp
from jax import lax
from jax.experimental import pallas as pl
from jax.experimental.pallas import tpu as pltpu
from jax.experimental.pallas import tpu_sc as plsc

_VOCAB = 100000
_B = 16384
_L = 200
_LOUT = _L + 2
_CLS = _VOCAB - 2
_SEP = _VOCAB - 1

_NW = 32
_ROWS_PER_W = _B // _NW
_R = 32
_NCHUNK = _ROWS_PER_W // _R
_CIN = _R * _L
_COUT = _R * _LOUT
_NVREG = _CIN // 16


def _sc_body(codes_hbm, table_hbm, out_hbm, table_v, codes_v, out_v, oidx_v):
    wid = lax.axis_index("s") * 2 + lax.axis_index("c")
    pltpu.sync_copy(table_hbm, table_v)

    lane = lax.iota(jnp.int32, 16)

    def oidx_body(j, carry):
        p = lane + j * 16
        oidx_v[pl.ds(j * 16, 16)] = p + 1 + 2 * lax.div(p, _L)
        return carry

    lax.fori_loop(0, _NVREG, oidx_body, 0)

    cls_v = jnp.full((16,), _CLS, jnp.int32)
    sep_v = jnp.full((16,), _SEP, jnp.int32)

    def chunk_body(c, carry):
        row0 = wid * _ROWS_PER_W + c * _R
        pltpu.sync_copy(codes_hbm.at[pl.ds(row0 * _L, _CIN)], codes_v)

        def gbody(j, gc):
            b = j * 16
            idx = codes_v[pl.ds(b, 16)]
            oidx = oidx_v[pl.ds(b, 16)]
            vals = plsc.load_gather(table_v, [idx])
            plsc.store_scatter(out_v, [oidx], vals)
            return gc

        lax.fori_loop(0, _NVREG, gbody, 0)

        for h in range(_R // 16):
            r16 = (lane + h * 16) * _LOUT
            plsc.store_scatter(out_v, [r16], cls_v)
            plsc.store_scatter(out_v, [r16 + (_LOUT - 1)], sep_v)

        pltpu.sync_copy(out_v, out_hbm.at[pl.ds(row0 * _LOUT, _COUT)])
        return carry

    lax.fori_loop(0, _NCHUNK, chunk_body, 0)


def kernel(char_codes, char_to_id):
    codes_flat = char_codes.reshape(-1)
    mesh = plsc.VectorSubcoreMesh(core_axis_name="c", subcore_axis_name="s")
    k = functools.partial(pl.kernel,
                          mesh=mesh,
                          out_type=jax.ShapeDtypeStruct((_B * _LOUT,), jnp.int32),
                          scratch_types=[
                              pltpu.VMEM((_VOCAB,), jnp.int32),
                              pltpu.VMEM((_CIN,), jnp.int32),
                              pltpu.VMEM((_COUT,), jnp.int32),
                              pltpu.VMEM((_CIN,), jnp.int32),
                          ],
                          compiler_params=pltpu.CompilerParams(
                              needs_layout_passes=False))(_sc_body)
    tok_flat = k(codes_flat, char_to_id)
    token_ids = tok_flat.reshape(_B, _LOUT)
    segment_ids = jnp.zeros_like(token_ids)
    return (token_ids, segment_ids)


# speedup vs baseline: 212.4570x; 1.3434x over previous
import functools

import jax
import jax.numpy as jnp
from jax import lax
from jax.experimental import pallas as pl
from jax.experimental.pallas import tpu as pltpu
from jax.experimental.pallas import tpu_sc as plsc

_VOCAB = 100000
_B = 16384
_L = 200
_LOUT = _L + 2
_CLS = _VOCAB - 2
_SEP = _VOCAB - 1

_NW = 32
_ROWS_PER_W = _B // _NW
_R = 32
_NCHUNK = _ROWS_PER_W // _R
_CIN = _R * _L
_COUT = _R * _LOUT
_NVREG = _CIN // 16


def _sc_body(codes_hbm, table_hbm, out_hbm, table_v, codes_v, out_v, oidx_v):
    wid = lax.axis_index("s") * 2 + lax.axis_index("c")
    pltpu.sync_copy(table_hbm, table_v)

    lane = lax.iota(jnp.int32, 16)

    @plsc.parallel_loop(0, _NVREG, unroll=4)
    def _(j):
        p = lane + j * 16
        oidx_v[pl.ds(j * 16, 16)] = p + 1 + 2 * lax.div(p, _L)

    cls_v = jnp.full((16,), _CLS, jnp.int32)
    sep_v = jnp.full((16,), _SEP, jnp.int32)

    def chunk_body(c, carry):
        row0 = wid * _ROWS_PER_W + c * _R
        pltpu.sync_copy(codes_hbm.at[pl.ds(row0 * _L, _CIN)], codes_v)

        @plsc.parallel_loop(0, _CIN, 16, unroll=8)
        def _(b):
            idx = codes_v[pl.ds(b, 16)]
            oidx = oidx_v[pl.ds(b, 16)]
            vals = plsc.load_gather(table_v, [idx])
            plsc.store_scatter(out_v, [oidx], vals)

        for h in range(_R // 16):
            r16 = (lane + h * 16) * _LOUT
            plsc.store_scatter(out_v, [r16], cls_v)
            plsc.store_scatter(out_v, [r16 + (_LOUT - 1)], sep_v)

        pltpu.sync_copy(out_v, out_hbm.at[pl.ds(row0 * _LOUT, _COUT)])
        return carry

    lax.fori_loop(0, _NCHUNK, chunk_body, 0)


def kernel(char_codes, char_to_id):
    codes_flat = char_codes.reshape(-1)
    mesh = plsc.VectorSubcoreMesh(core_axis_name="c", subcore_axis_name="s")
    k = functools.partial(pl.kernel,
                          mesh=mesh,
                          out_type=jax.ShapeDtypeStruct((_B * _LOUT,), jnp.int32),
                          scratch_types=[
                              pltpu.VMEM((_VOCAB,), jnp.int32),
                              pltpu.VMEM((_CIN,), jnp.int32),
                              pltpu.VMEM((_COUT,), jnp.int32),
                              pltpu.VMEM((_CIN,), jnp.int32),
                          ],
                          compiler_params=pltpu.CompilerParams(
                              needs_layout_passes=False))(_sc_body)
    tok_flat = k(codes_flat, char_to_id)
    token_ids = tok_flat.reshape(_B, _LOUT)
    segment_ids = jnp.zeros_like(token_ids)
    return (token_ids, segment_ids)


# speedup vs baseline: 224.5582x; 1.0570x over previous
import functools

import jax
import jax.numpy as jnp
from jax import lax
from jax.experimental import pallas as pl
from jax.experimental.pallas import tpu as pltpu
from jax.experimental.pallas import tpu_sc as plsc

_VOCAB = 100000
_B = 16384
_L = 200
_LOUT = _L + 2
_CLS = _VOCAB - 2
_SEP = _VOCAB - 1

_NW = 32
_ROWS_PER_W = _B // _NW
_R = 32
_NCHUNK = _ROWS_PER_W // _R
_NFULL = _L // 16
_TAIL = _L - 16


def _sc_body(codes_hbm, table_hbm, out_hbm, table_v, codes_v, out_v):
    wid = lax.axis_index("s") * 2 + lax.axis_index("c")
    pltpu.sync_copy(table_hbm, table_v)

    lane = lax.iota(jnp.int32, 16)
    cls_v = jnp.full((16,), _CLS, jnp.int32)
    sep_v = jnp.full((16,), _SEP, jnp.int32)
    col0 = jnp.zeros((16,), jnp.int32)
    col_last = jnp.full((16,), _LOUT - 1, jnp.int32)

    def chunk_body(c, carry):
        row0 = wid * _ROWS_PER_W + c * _R
        pltpu.sync_copy(codes_hbm.at[pl.ds(row0, _R), :], codes_v)

        @plsc.parallel_loop(0, _R, unroll=2)
        def _(r):
            row_vec = jnp.full((16,), 0, jnp.int32) + r
            for j in range(_NFULL):
                idx = codes_v[r, pl.ds(j * 16, 16)]
                vals = plsc.load_gather(table_v, [idx])
                plsc.store_scatter(out_v, [row_vec, lane + (1 + j * 16)], vals)
            idx = codes_v[r, pl.ds(_TAIL, 16)]
            vals = plsc.load_gather(table_v, [idx])
            plsc.store_scatter(out_v, [row_vec, lane + (1 + _TAIL)], vals)

        for h in range(_R // 16):
            rows = lane + h * 16
            plsc.store_scatter(out_v, [rows, col0], cls_v)
            plsc.store_scatter(out_v, [rows, col_last], sep_v)

        pltpu.sync_copy(out_v, out_hbm.at[pl.ds(row0, _R), :])
        return carry

    lax.fori_loop(0, _NCHUNK, chunk_body, 0)


def kernel(char_codes, char_to_id):
    mesh = plsc.VectorSubcoreMesh(core_axis_name="c", subcore_axis_name="s")
    k = functools.partial(pl.kernel,
                          mesh=mesh,
                          out_type=jax.ShapeDtypeStruct((_B, _LOUT), jnp.int32),
                          scratch_types=[
                              pltpu.VMEM((_VOCAB,), jnp.int32),
                              pltpu.VMEM((_R, _L), jnp.int32),
                              pltpu.VMEM((_R, _LOUT), jnp.int32),
                          ],
                          compiler_params=pltpu.CompilerParams(
                              needs_layout_passes=False,
                              use_tc_tiling_on_sc=False))(_sc_body)
    token_ids = k(char_codes, char_to_id)
    segment_ids = jnp.zeros_like(token_ids)
    return (token_ids, segment_ids)
